# initial kernel scaffold (unmeasured)
import jax
import jax.numpy as jnp
from jax import lax
from jax.experimental import pallas as pl
from jax.experimental.pallas import tpu as pltpu

N_DEV = 8
N_TOK = 2048
D = 1024
N_EXP = 64
EXP_PER_DEV = 8
CAP = 25
CAP_PAD = 32
S = EXP_PER_DEV * CAP_PAD
TOK_PER_DEV = N_TOK // N_DEV


def kernel(x, router_W, route_idx, expert_W):
    del router_W

    def body(x_ref, idx_ref, w_hbm, out_ref,
             g_ref, y_ref, p_ref, recv_ref, w_vmem,
             w_sems, send_sems, recv_sems):
        my = lax.axis_index("i")

        barrier_sem = pltpu.get_barrier_semaphore()
        for off in range(1, N_DEV):
            pl.semaphore_signal(
                barrier_sem, inc=1,
                device_id=((my + off) % N_DEV,),
                device_id_type=pl.DeviceIdType.MESH,
            )
        pl.semaphore_wait(barrier_sem, N_DEV - 1)

        def w_copy(k, slot):
            return pltpu.make_async_copy(
                w_hbm.at[k], w_vmem.at[slot], w_sems.at[slot])

        w_copy(0, 0).start()

        e = idx_ref[:, :]
        exp_iota = lax.broadcasted_iota(jnp.int32, (N_TOK, N_EXP), 1)
        m = (e == exp_iota).astype(jnp.float32)
        c = m
        sh = 1
        while sh < N_TOK:
            c = c + jnp.concatenate(
                [jnp.zeros((sh, N_EXP), jnp.float32), c[:-sh, :]], axis=0)
            sh *= 2
        csel = jnp.sum(jnp.where(m > 0.5, c, 0.0), axis=1, keepdims=True)
        rank = csel.astype(jnp.int32) - 1
        le = e - my * EXP_PER_DEV
        ok = (le >= 0) & (le < EXP_PER_DEV) & (rank < CAP)
        slot = jnp.where(ok, le * CAP_PAD + rank, -1)
        slot_iota = lax.broadcasted_iota(jnp.int32, (N_TOK, S), 1)
        g = (slot == slot_iota).astype(jnp.float32)
        g_ref[:, :] = g

        xg = lax.dot_general(
            g, x_ref[:, :],
            dimension_numbers=(((0,), (0,)), ((), ())),
            preferred_element_type=jnp.float32,
        )

        for k in range(EXP_PER_DEV):
            if k + 1 < EXP_PER_DEV:
                w_copy(k + 1, (k + 1) % 2).start()
            w_copy(k, k % 2).wait()
            y_ref[k * CAP_PAD:(k + 1) * CAP_PAD, :] = jnp.dot(
                xg[k * CAP_PAD:(k + 1) * CAP_PAD, :],
                w_vmem[k % 2],
                preferred_element_type=jnp.float32,
            )

        p_ref[:, :] = jnp.dot(g_ref[:, :], y_ref[:, :],
                              preferred_element_type=jnp.float32)

        rdmas = []
        for off in range(1, N_DEV):
            dst = (my + off) % N_DEV
            rdma = pltpu.make_async_remote_copy(
                src_ref=p_ref.at[pl.ds(dst * TOK_PER_DEV, TOK_PER_DEV), :],
                dst_ref=recv_ref.at[my],
                send_sem=send_sems.at[off - 1],
                recv_sem=recv_sems.at[my],
                device_id=(dst,),
                device_id_type=pl.DeviceIdType.MESH,
            )
            rdma.start()
            rdmas.append(rdma)

        recv_ref[pl.ds(my, 1), :, :] = (
            p_ref[pl.ds(my * TOK_PER_DEV, TOK_PER_DEV), :][None]
        )

        for off in range(1, N_DEV):
            src_dev = (my + off) % N_DEV
            pltpu.make_async_remote_copy(
                src_ref=p_ref.at[pl.ds(0, TOK_PER_DEV), :],
                dst_ref=recv_ref.at[src_dev],
                send_sem=send_sems.at[0],
                recv_sem=recv_sems.at[src_dev],
                device_id=(src_dev,),
                device_id_type=pl.DeviceIdType.MESH,
            ).wait_recv()

        acc = recv_ref[0, :, :]
        for q in range(1, N_DEV):
            acc = acc + recv_ref[q, :, :]
        out_ref[:, :] = acc

        for rdma in rdmas:
            rdma.wait_send()

    return pl.pallas_call(
        body,
        out_shape=jax.ShapeDtypeStruct((TOK_PER_DEV, D), jnp.float32),
        in_specs=[
            pl.BlockSpec(memory_space=pltpu.VMEM),
            pl.BlockSpec(memory_space=pltpu.VMEM),
            pl.BlockSpec(memory_space=pltpu.ANY),
        ],
        out_specs=pl.BlockSpec(memory_space=pltpu.VMEM),
        scratch_shapes=[
            pltpu.VMEM((N_TOK, S), jnp.float32),
            pltpu.VMEM((S, D), jnp.float32),
            pltpu.VMEM((N_TOK, D), jnp.float32),
            pltpu.VMEM((N_DEV, TOK_PER_DEV, D), jnp.float32),
            pltpu.VMEM((2, D, D), jnp.float32),
            pltpu.SemaphoreType.DMA((2,)),
            pltpu.SemaphoreType.DMA((N_DEV - 1,)),
            pltpu.SemaphoreType.DMA((N_DEV,)),
        ],
        compiler_params=pltpu.CompilerParams(collective_id=0),
    )(x, route_idx, expert_W)


# baseline (device time: 93021 ns/iter reference)
import jax
import jax.numpy as jnp
from jax import lax
from jax.experimental import pallas as pl
from jax.experimental.pallas import tpu as pltpu

N_DEV = 8
N_TOK = 2048
D = 1024
N_EXP = 64
EXP_PER_DEV = 8
CAP = 25
CAP_PAD = 32
S = EXP_PER_DEV * CAP_PAD
TOK_PER_DEV = N_TOK // N_DEV


def kernel(x, router_W, route_idx, expert_W):
    del router_W

    def body(x_ref, idx_ref, w_hbm, out_ref,
             g_ref, y_ref, p_ref, recv_ref, w_vmem,
             w_sems, send_sems, recv_sems):
        my = lax.axis_index("i")

        barrier_sem = pltpu.get_barrier_semaphore()
        for off in range(1, N_DEV):
            pl.semaphore_signal(
                barrier_sem, inc=1,
                device_id=((my + off) % N_DEV,),
                device_id_type=pl.DeviceIdType.MESH,
            )
        pl.semaphore_wait(barrier_sem, N_DEV - 1)

        def w_copy(k, slot):
            return pltpu.make_async_copy(
                w_hbm.at[k], w_vmem.at[slot], w_sems.at[slot])

        w_copy(0, 0).start()

        e = idx_ref[:, :]
        exp_iota = lax.broadcasted_iota(jnp.int32, (N_TOK, N_EXP), 1)
        m = (e == exp_iota).astype(jnp.float32)
        c = m
        sh = 1
        while sh < N_TOK:
            c = c + jnp.concatenate(
                [jnp.zeros((sh, N_EXP), jnp.float32), c[:-sh, :]], axis=0)
            sh *= 2
        csel = jnp.sum(jnp.where(m > 0.5, c, 0.0), axis=1, keepdims=True)
        rank = csel.astype(jnp.int32) - 1
        le = e - my * EXP_PER_DEV
        ok = (le >= 0) & (le < EXP_PER_DEV) & (rank < CAP)
        slot = jnp.where(ok, le * CAP_PAD + rank, -1)
        slot_iota = lax.broadcasted_iota(jnp.int32, (N_TOK, S), 1)
        g = (slot == slot_iota).astype(jnp.float32)
        g_ref[:, :] = g

        xg = lax.dot_general(
            g, x_ref[:, :],
            dimension_numbers=(((0,), (0,)), ((), ())),
            preferred_element_type=jnp.float32,
        )

        for k in range(EXP_PER_DEV):
            if k + 1 < EXP_PER_DEV:
                w_copy(k + 1, (k + 1) % 2).start()
            w_copy(k, k % 2).wait()
            y_ref[k * CAP_PAD:(k + 1) * CAP_PAD, :] = jnp.dot(
                xg[k * CAP_PAD:(k + 1) * CAP_PAD, :],
                w_vmem[k % 2],
                preferred_element_type=jnp.float32,
            )

        p_ref[:, :] = jnp.dot(g_ref[:, :], y_ref[:, :],
                              preferred_element_type=jnp.float32)

        rdmas = []
        for off in range(1, N_DEV):
            dst = (my + off) % N_DEV
            rdma = pltpu.make_async_remote_copy(
                src_ref=p_ref.at[pl.ds(dst * TOK_PER_DEV, TOK_PER_DEV), :],
                dst_ref=recv_ref.at[my],
                send_sem=send_sems.at[off - 1],
                recv_sem=recv_sems.at[my],
                device_id=(dst,),
                device_id_type=pl.DeviceIdType.MESH,
            )
            rdma.start()
            rdmas.append(rdma)

        recv_ref[pl.ds(my, 1), :, :] = (
            p_ref[pl.ds(my * TOK_PER_DEV, TOK_PER_DEV), :][None]
        )

        for off in range(1, N_DEV):
            src_dev = (my + off) % N_DEV
            pltpu.make_async_remote_copy(
                src_ref=p_ref.at[pl.ds(0, TOK_PER_DEV), :],
                dst_ref=recv_ref.at[src_dev],
                send_sem=send_sems.at[0],
                recv_sem=recv_sems.at[src_dev],
                device_id=(src_dev,),
                device_id_type=pl.DeviceIdType.MESH,
            ).wait_recv()

        acc = recv_ref[0, :, :]
        for q in range(1, N_DEV):
            acc = acc + recv_ref[q, :, :]
        out_ref[:, :] = acc

        for rdma in rdmas:
            rdma.wait_send()

    return pl.pallas_call(
        body,
        out_shape=jax.ShapeDtypeStruct((TOK_PER_DEV, D), jnp.float32),
        in_specs=[
            pl.BlockSpec(memory_space=pltpu.VMEM),
            pl.BlockSpec(memory_space=pltpu.VMEM),
            pl.BlockSpec(memory_space=pltpu.MemorySpace.HBM),
        ],
        out_specs=pl.BlockSpec(memory_space=pltpu.VMEM),
        scratch_shapes=[
            pltpu.VMEM((N_TOK, S), jnp.float32),
            pltpu.VMEM((S, D), jnp.float32),
            pltpu.VMEM((N_TOK, D), jnp.float32),
            pltpu.VMEM((N_DEV, TOK_PER_DEV, D), jnp.float32),
            pltpu.VMEM((2, D, D), jnp.float32),
            pltpu.SemaphoreType.DMA((2,)),
            pltpu.SemaphoreType.DMA((N_DEV - 1,)),
            pltpu.SemaphoreType.DMA((N_DEV,)),
        ],
        compiler_params=pltpu.CompilerParams(collective_id=0),
    )(x, route_idx, expert_W)


# device time: 34591 ns/iter; 2.6892x vs baseline; 2.6892x over previous
import jax
import jax.numpy as jnp
from jax import lax
from jax.experimental import pallas as pl
from jax.experimental.pallas import tpu as pltpu

N_DEV = 8
N_TOK = 2048
D = 1024
N_EXP = 64
EXP_PER_DEV = 8
CAP = 25
CAP_PAD = 32
S = EXP_PER_DEV * CAP_PAD
TOK_PER_DEV = N_TOK // N_DEV
N_WBUF = 4
PC = 224
BLK = 32
NBLK = PC // BLK


_CS_BLK = 256


def _cumsum_tokens(v):
    n, w = v.shape
    tri = (
        lax.broadcasted_iota(jnp.int32, (_CS_BLK, _CS_BLK), 0)
        >= lax.broadcasted_iota(jnp.int32, (_CS_BLK, _CS_BLK), 1)
    ).astype(jnp.bfloat16)
    out_blocks = []
    run = jnp.zeros((1, w), jnp.float32)
    for b in range(n // _CS_BLK):
        blk = v[b * _CS_BLK:(b + 1) * _CS_BLK, :]
        cw = lax.dot_general(
            tri, blk, dimension_numbers=(((1,), (0,)), ((), ())),
            preferred_element_type=jnp.float32,
        )
        out_blocks.append(cw + run)
        run = run + cw[_CS_BLK - 1:_CS_BLK, :]
    return jnp.concatenate(out_blocks, axis=0)


def kernel(x, router_W, route_idx, expert_W):
    del router_W

    def body(x_ref, idx_ref, w_hbm, out_ref,
             y_ref, comp_ref, recv_ref, tv_ref, w_vmem,
             w_sems, send_sems, recv_sems):
        my = lax.axis_index("i")

        def w_copy(k, slot):
            return pltpu.make_async_copy(
                w_hbm.at[k], w_vmem.at[slot], w_sems.at[slot])

        for k in range(N_WBUF):
            w_copy(k, k).start()

        recv_ref[:, :, :] = jnp.zeros((N_DEV, PC, D), jnp.bfloat16)

        barrier_sem = pltpu.get_barrier_semaphore()
        for off in range(1, N_DEV):
            pl.semaphore_signal(
                barrier_sem, inc=1,
                device_id=((my + off) % N_DEV,),
                device_id_type=pl.DeviceIdType.MESH,
            )
        pl.semaphore_wait(barrier_sem, N_DEV - 1)

        e = idx_ref[:, :]
        exp_iota = lax.broadcasted_iota(jnp.int32, (N_TOK, N_EXP), 1)
        m = (e == exp_iota).astype(jnp.bfloat16)
        c = _cumsum_tokens(m)
        csel = jnp.sum(jnp.where(m > 0.5, c, 0.0), axis=1, keepdims=True)
        acc = csel <= float(CAP)
        rank = csel.astype(jnp.int32) - 1
        le = e - my * EXP_PER_DEV
        ok = (le >= 0) & (le < EXP_PER_DEV) & acc
        slot = jnp.where(ok, le * CAP_PAD + rank, -1)
        slot_iota = lax.broadcasted_iota(jnp.int32, (N_TOK, S), 1)
        g = (slot == slot_iota).astype(jnp.bfloat16)

        tok = lax.broadcasted_iota(jnp.int32, (N_TOK, 1), 0)
        owner = tok // TOK_PER_DEV
        sender = e // EXP_PER_DEV
        qo = sender * N_DEV + owner
        qom = ((qo == exp_iota) & acc).astype(jnp.bfloat16)
        c2 = _cumsum_tokens(qom)
        pos = jnp.sum(jnp.where(qom > 0.5, c2, 0.0), axis=1,
                      keepdims=True) - 1.0

        tv_ref[:, :] = jnp.concatenate(
            [pos, sender.astype(jnp.float32),
             jnp.where(acc, 1.0, 0.0),
             jnp.zeros((N_TOK, 5), jnp.float32)], axis=1)

        xg = lax.dot_general(
            g, x_ref[:, :].astype(jnp.bfloat16),
            dimension_numbers=(((0,), (0,)), ((), ())),
            preferred_element_type=jnp.float32,
        )

        xgb = xg.astype(jnp.bfloat16)
        for k in range(EXP_PER_DEV):
            if k + N_WBUF < EXP_PER_DEV:
                w_copy(k + N_WBUF, (k + N_WBUF) % N_WBUF).start()
            w_copy(k, k % N_WBUF).wait()
            y_ref[k * CAP_PAD:(k + 1) * CAP_PAD, :] = jnp.dot(
                xgb[k * CAP_PAD:(k + 1) * CAP_PAD, :],
                w_vmem[k % N_WBUF].astype(jnp.bfloat16),
                preferred_element_type=jnp.float32,
            ).astype(jnp.bfloat16)

        z = jnp.concatenate(
            [pos.astype(jnp.bfloat16), owner.astype(jnp.bfloat16),
             jnp.ones((N_TOK, 1), jnp.bfloat16),
             jnp.zeros((N_TOK, 5), jnp.bfloat16)], axis=1)
        sm = lax.dot_general(
            g, z, dimension_numbers=(((0,), (0,)), ((), ())),
            preferred_element_type=jnp.float32,
        )
        spos, sown, socc = sm[:, 0:1], sm[:, 1:2], sm[:, 2:3]
        ci = jnp.where(socc > 0.5, sown * PC + spos, -1.0).astype(jnp.int32)
        ci_iota = lax.broadcasted_iota(jnp.int32, (S, N_DEV * PC), 1)
        ch = (ci == ci_iota).astype(jnp.bfloat16)
        comp = lax.dot_general(
            ch, y_ref[:, :],
            dimension_numbers=(((0,), (0,)), ((), ())),
            preferred_element_type=jnp.float32,
        )
        comp_ref[:, :] = comp.astype(jnp.bfloat16)

        def pair_cnt(src_dev, dst_dev):
            return jnp.sum(jnp.where(
                (sender == src_dev) & (owner == dst_dev) & acc, 1.0, 0.0))

        def send_rdma(p, off, b):
            return pltpu.make_async_remote_copy(
                src_ref=comp_ref.at[pl.ds(p * PC + b * BLK, BLK), :],
                dst_ref=recv_ref.at[my, pl.ds(b * BLK, BLK), :],
                send_sem=send_sems.at[off - 1, b],
                recv_sem=recv_sems.at[my, b],
                device_id=(p,),
                device_id_type=pl.DeviceIdType.MESH,
            )

        def recv_rdma(q, b):
            return pltpu.make_async_remote_copy(
                src_ref=comp_ref.at[pl.ds(0, BLK), :],
                dst_ref=recv_ref.at[q, pl.ds(b * BLK, BLK), :],
                send_sem=send_sems.at[0, 0],
                recv_sem=recv_sems.at[q, b],
                device_id=(q,),
                device_id_type=pl.DeviceIdType.MESH,
            )

        send_cnts = []
        for off in range(1, N_DEV):
            p = (my + off) % N_DEV
            cnt = pair_cnt(my, p)
            send_cnts.append((p, cnt))
            send_rdma(p, off, 0).start()
            for b in range(1, NBLK):
                @pl.when(cnt > float(BLK * b))
                def _(p=p, off=off, b=b):
                    send_rdma(p, off, b).start()

        recv_ref[pl.ds(my, 1), :, :] = (
            comp_ref[pl.ds(my * PC, PC), :][None]
        )

        sl = tv_ref[pl.ds(my * TOK_PER_DEV, TOK_PER_DEV), :]
        rpos, rsend, racc = sl[:, 0:1], sl[:, 1:2], sl[:, 2:3]
        cr = jnp.where(racc > 0.5, rsend * PC + rpos, -1.0).astype(jnp.int32)
        cr_iota = lax.broadcasted_iota(
            jnp.int32, (TOK_PER_DEV, N_DEV * PC), 1)
        rh = (cr == cr_iota).astype(jnp.bfloat16)

        for off in range(1, N_DEV):
            q = (my + off) % N_DEV
            cnt = pair_cnt(q, my)
            recv_rdma(q, 0).wait_recv()
            for b in range(1, NBLK):
                @pl.when(cnt > float(BLK * b))
                def _(q=q, b=b):
                    recv_rdma(q, b).wait_recv()

        recv_all = recv_ref[:, :, :].reshape(N_DEV * PC, D)
        out_ref[:, :] = jnp.dot(rh, recv_all,
                                preferred_element_type=jnp.float32)

        for (p, cnt), off in zip(send_cnts, range(1, N_DEV)):
            send_rdma(p, off, 0).wait_send()
            for b in range(1, NBLK):
                @pl.when(cnt > float(BLK * b))
                def _(p=p, off=off, b=b):
                    send_rdma(p, off, b).wait_send()

    return pl.pallas_call(
        body,
        out_shape=jax.ShapeDtypeStruct((TOK_PER_DEV, D), jnp.float32),
        in_specs=[
            pl.BlockSpec(memory_space=pltpu.VMEM),
            pl.BlockSpec(memory_space=pltpu.VMEM),
            pl.BlockSpec(memory_space=pltpu.MemorySpace.HBM),
        ],
        out_specs=pl.BlockSpec(memory_space=pltpu.VMEM),
        scratch_shapes=[
            pltpu.VMEM((S, D), jnp.bfloat16),
            pltpu.VMEM((N_DEV * PC, D), jnp.bfloat16),
            pltpu.VMEM((N_DEV, PC, D), jnp.bfloat16),
            pltpu.VMEM((N_TOK, 8), jnp.float32),
            pltpu.VMEM((N_WBUF, D, D), jnp.float32),
            pltpu.SemaphoreType.DMA((N_WBUF,)),
            pltpu.SemaphoreType.DMA((N_DEV - 1, NBLK)),
            pltpu.SemaphoreType.DMA((N_DEV, NBLK)),
        ],
        compiler_params=pltpu.CompilerParams(collective_id=0),
    )(x, route_idx, expert_W)
